# Initial kernel scaffold; baseline (speedup 1.0000x reference)
import jax
import jax.numpy as jnp
from jax.experimental import pallas as pl


def kernel(x, edge_index, W0, b0, lng0, lnb0, W1, b1, lng1, lnb1, W2, b2, lng2, lnb2, W3, b3, lng3, lnb3, mW0, mb0, mW1, mb1, mW2, mb2, mg0, mlb0, mg1, mlb1):
    raise NotImplementedError("write your pallas kernel here")


# speedup vs baseline: 6.7249x; 6.7249x over previous
import functools

import jax
import jax.numpy as jnp
from jax import lax
from jax.experimental import pallas as pl
from jax.experimental.pallas import tpu as pltpu
from jax.experimental.pallas import tpu_sc as plsc

NC, NS, L = 2, 16, 16
NW = NC * NS


def _sc_mesh():
    return plsc.VectorSubcoreMesh(core_axis_name="c", subcore_axis_name="s",
                                  num_cores=NC, num_subcores=NS)


_SC_PARAMS = pltpu.CompilerParams(needs_layout_passes=False)


def _ln(y, g, b):
    m = jnp.mean(y, axis=-1, keepdims=True)
    v = jnp.mean((y - m) ** 2, axis=-1, keepdims=True)
    return (y - m) * lax.rsqrt(v + 1e-5) * g + b


def _make_deg(E, N):
    EPW = E // NW

    @functools.partial(
        pl.kernel,
        out_type=jax.ShapeDtypeStruct((NW * N,), jnp.float32),
        mesh=_sc_mesh(),
        compiler_params=_SC_PARAMS,
        scratch_types=[
            pltpu.VMEM((EPW,), jnp.int32),
            pltpu.VMEM((N,), jnp.float32),
        ],
    )
    def k(col_hbm, out_hbm, colv, hist):
        c = lax.axis_index("c")
        s = lax.axis_index("s")
        wid = s * NC + c
        zero16 = jnp.zeros((L,), jnp.float32)
        ones16 = jnp.ones((L,), jnp.float32)

        def zb(i, carry):
            hist[pl.ds(i * L, L)] = zero16
            return carry

        lax.fori_loop(0, N // L, zb, 0)
        pltpu.sync_copy(col_hbm.at[pl.ds(wid * EPW, EPW)], colv)

        def body(j, carry):
            idx = colv[pl.ds(j * L, L)]
            plsc.addupdate_scatter(hist, [idx], ones16)
            return carry

        lax.fori_loop(0, EPW // L, body, 0)
        pltpu.sync_copy(hist, out_hbm.at[pl.ds(wid * N, N)])

    return k


def _make_scatter(E, N, H):
    EPW = E // NW
    CH = 80
    NCH = EPW // CH
    ZR = 128
    NP = -(-N // (ZR * NS)) * (ZR * NS)
    NPT = NP // NS

    @functools.partial(
        pl.kernel,
        out_type=jax.ShapeDtypeStruct((NC, NP, H), jnp.float32),
        mesh=_sc_mesh(),
        compiler_params=_SC_PARAMS,
        scratch_types=[
            pltpu.VMEM((CH,), jnp.int32),
            pltpu.VMEM((CH,), jnp.int32),
            pltpu.VMEM((CH, H), jnp.float32),
            pltpu.VMEM((ZR, H), jnp.float32),
            pltpu.VMEM_SHARED((NP, H), jnp.float32),
            pltpu.SemaphoreType.DMA,
        ],
    )
    def k(g_hbm, row_hbm, col_hbm, out_hbm, rowv, colv, rows, zbuf, acc, sem):
        c = lax.axis_index("c")
        s = lax.axis_index("s")
        wid = s * NC + c
        zero16 = jnp.zeros((L,), jnp.float32)
        hl = H // L

        def zb(i, carry):
            zbuf[i // hl, pl.ds((i % hl) * L, L)] = zero16
            return carry

        lax.fori_loop(0, ZR * hl, zb, 0)
        for j in range(NPT // ZR):
            pltpu.sync_copy(zbuf, acc.at[pl.ds(s * NPT + j * ZR, ZR)])
        plsc.subcore_barrier()

        base = wid * EPW

        def body(t, carry):
            e0 = base + t * CH
            pltpu.sync_copy(row_hbm.at[pl.ds(e0, CH)], rowv)
            pltpu.sync_copy(col_hbm.at[pl.ds(e0, CH)], colv)
            pltpu.async_copy(g_hbm.at[rowv], rows, sem).wait()
            pltpu.sync_copy(rows, acc.at[colv], add=True)
            return carry

        lax.fori_loop(0, NCH, body, 0)
        plsc.subcore_barrier()
        pltpu.sync_copy(acc.at[pl.ds(s * NPT, NPT)],
                        out_hbm.at[c, pl.ds(s * NPT, NPT)])

    return k


def _make_gather_he(E, N, H):
    EPW = E // NW
    CH = 80
    NCH = EPW // CH

    @functools.partial(
        pl.kernel,
        out_type=jax.ShapeDtypeStruct((2, E, H), jnp.float32),
        mesh=_sc_mesh(),
        compiler_params=_SC_PARAMS,
        scratch_types=[
            pltpu.VMEM((CH,), jnp.int32),
            pltpu.VMEM((CH,), jnp.int32),
            pltpu.VMEM((CH, H), jnp.float32),
            pltpu.VMEM((CH, H), jnp.float32),
            pltpu.SemaphoreType.DMA,
        ],
    )
    def k(x4_hbm, src_hbm, dst_hbm, out_hbm, srcv, dstv, bufs, bufd, sem):
        c = lax.axis_index("c")
        s = lax.axis_index("s")
        wid = s * NC + c
        base = wid * EPW

        def body(t, carry):
            e0 = base + t * CH
            pltpu.sync_copy(src_hbm.at[pl.ds(e0, CH)], srcv)
            pltpu.sync_copy(dst_hbm.at[pl.ds(e0, CH)], dstv)
            pltpu.async_copy(x4_hbm.at[srcv], bufs, sem).wait()
            pltpu.sync_copy(bufs, out_hbm.at[0, pl.ds(e0, CH)])
            pltpu.async_copy(x4_hbm.at[dstv], bufd, sem).wait()
            pltpu.sync_copy(bufd, out_hbm.at[1, pl.ds(e0, CH)])
            return carry

        lax.fori_loop(0, NCH, body, 0)

    return k


NB = 1000
EB = 2000


def _prologue_call(degT, x, W0):
    N, D = x.shape
    H = W0.shape[1]

    def body(degT_ref, x_ref, w_ref, dinv_ref, g_ref):
        deg = jnp.sum(degT_ref[...], axis=-1, keepdims=True) + 1.0
        dinv = lax.rsqrt(deg)
        h = jnp.dot(x_ref[...], w_ref[...], preferred_element_type=jnp.float32)
        dinv_ref[...] = dinv
        g_ref[...] = h * dinv

    return pl.pallas_call(
        body,
        grid=(N // NB,),
        in_specs=[
            pl.BlockSpec((NB, NW), lambda i: (i, 0)),
            pl.BlockSpec((NB, D), lambda i: (i, 0)),
            pl.BlockSpec((D, H), lambda i: (0, 0)),
        ],
        out_specs=[
            pl.BlockSpec((NB, 1), lambda i: (i, 0)),
            pl.BlockSpec((NB, H), lambda i: (i, 0)),
        ],
        out_shape=[
            jax.ShapeDtypeStruct((N, 1), jnp.float32),
            jax.ShapeDtypeStruct((N, H), jnp.float32),
        ],
    )(degT, x, W0)


def _layer_call(acc, g, dinv, res, b, lng, lnb, Wn):
    N, H = g.shape
    with_res = res is not None
    with_next = Wn is not None

    def body(*refs):
        it = iter(refs)
        acc_ref = next(it)
        g_ref = next(it)
        dinv_ref = next(it)
        res_ref = next(it) if with_res else None
        b_ref = next(it)
        lng_ref = next(it)
        lnb_ref = next(it)
        w_ref = next(it) if with_next else None
        xn_ref = next(it)
        gn_ref = next(it) if with_next else None

        dinv = dinv_ref[...]
        y = (acc_ref[0] + acc_ref[1] + g_ref[...]) * dinv + b_ref[...]
        y = jnp.maximum(y, 0.0)
        y = _ln(y, lng_ref[...], lnb_ref[...])
        if with_res:
            y = y + res_ref[...]
        xn_ref[...] = y
        if with_next:
            gn_ref[...] = jnp.dot(
                y, w_ref[...], preferred_element_type=jnp.float32) * dinv

    in_specs = [pl.BlockSpec((2, NB, H), lambda i: (0, i, 0)),
                pl.BlockSpec((NB, H), lambda i: (i, 0)),
                pl.BlockSpec((NB, 1), lambda i: (i, 0))]
    ins = [acc, g, dinv]
    if with_res:
        in_specs.append(pl.BlockSpec((NB, H), lambda i: (i, 0)))
        ins.append(res)
    in_specs += [pl.BlockSpec((1, H), lambda i: (0, 0))] * 3
    ins += [b.reshape(1, H), lng.reshape(1, H), lnb.reshape(1, H)]
    if with_next:
        in_specs.append(pl.BlockSpec((H, H), lambda i: (0, 0)))
        ins.append(Wn)

    out_specs = [pl.BlockSpec((NB, H), lambda i: (i, 0))]
    out_shape = [jax.ShapeDtypeStruct((N, H), jnp.float32)]
    if with_next:
        out_specs.append(pl.BlockSpec((NB, H), lambda i: (i, 0)))
        out_shape.append(jax.ShapeDtypeStruct((N, H), jnp.float32))

    out = pl.pallas_call(
        body,
        grid=(N // NB,),
        in_specs=in_specs,
        out_specs=out_specs,
        out_shape=out_shape,
    )(*ins)
    return out if with_next else out[0]


def _mlp_call(he, mW0a, mW0b, mb0, mg0, mlb0, mW1, mb1, mg1, mlb1, mW2, mb2):
    _, E, H = he.shape
    F0 = mW0a.shape[1]
    F1 = mW1.shape[1]

    def body(he_ref, w0a_ref, w0b_ref, b0_ref, g0_ref, lb0_ref,
             w1_ref, b1_ref, g1_ref, lb1_ref, w2_ref, b2_ref, o_ref):
        z = (jnp.dot(he_ref[0], w0a_ref[...], preferred_element_type=jnp.float32)
             + jnp.dot(he_ref[1], w0b_ref[...], preferred_element_type=jnp.float32)
             + b0_ref[...])
        z = jnp.maximum(_ln(z, g0_ref[...], lb0_ref[...]), 0.0)
        t = jnp.dot(z, w1_ref[...], preferred_element_type=jnp.float32) + b1_ref[...]
        t = jnp.maximum(_ln(t, g1_ref[...], lb1_ref[...]), 0.0)
        o_ref[...] = (jnp.dot(t, w2_ref[...], preferred_element_type=jnp.float32)
                      + b2_ref[...])

    return pl.pallas_call(
        body,
        grid=(E // EB,),
        in_specs=[
            pl.BlockSpec((2, EB, H), lambda i: (0, i, 0)),
            pl.BlockSpec((H, F0), lambda i: (0, 0)),
            pl.BlockSpec((H, F0), lambda i: (0, 0)),
            pl.BlockSpec((1, F0), lambda i: (0, 0)),
            pl.BlockSpec((1, F0), lambda i: (0, 0)),
            pl.BlockSpec((1, F0), lambda i: (0, 0)),
            pl.BlockSpec((F0, F1), lambda i: (0, 0)),
            pl.BlockSpec((1, F1), lambda i: (0, 0)),
            pl.BlockSpec((1, F1), lambda i: (0, 0)),
            pl.BlockSpec((1, F1), lambda i: (0, 0)),
            pl.BlockSpec((F1, 1), lambda i: (0, 0)),
            pl.BlockSpec((1, 1), lambda i: (0, 0)),
        ],
        out_specs=[pl.BlockSpec((EB, 1), lambda i: (i, 0))],
        out_shape=[jax.ShapeDtypeStruct((E, 1), jnp.float32)],
    )(he, mW0a, mW0b, mb0.reshape(1, F0), mg0.reshape(1, F0),
      mlb0.reshape(1, F0), mW1, mb1.reshape(1, F1), mg1.reshape(1, F1),
      mlb1.reshape(1, F1), mW2, mb2.reshape(1, 1))[0]


def kernel(x, edge_index, W0, b0, lng0, lnb0, W1, b1, lng1, lnb1,
           W2, b2, lng2, lnb2, W3, b3, lng3, lnb3,
           mW0, mb0, mW1, mb1, mW2, mb2, mg0, mlb0, mg1, mlb1):
    N, D = x.shape
    H = W0.shape[1]
    E = edge_index.shape[1]

    ei = edge_index.astype(jnp.int32)
    row, col = ei[0], ei[1]

    deg_k = _make_deg(E, N)
    scat_k = _make_scatter(E, N, H)
    he_k = _make_gather_he(E, N, H)

    degs = deg_k(col).reshape(NW, N)
    dinv, g0 = _prologue_call(degs.T, x, W0)

    acc0 = scat_k(g0, row, col)
    x1, g1 = _layer_call(acc0, g0, dinv, None, b0, lng0, lnb0, W1)
    acc1 = scat_k(g1, row, col)
    x2, g2 = _layer_call(acc1, g1, dinv, x, b1, lng1, lnb1, W2)
    acc2 = scat_k(g2, row, col)
    x3, g3 = _layer_call(acc2, g2, dinv, None, b2, lng2, lnb2, W3)
    acc3 = scat_k(g3, row, col)
    x4 = _layer_call(acc3, g3, dinv, x2, b3, lng3, lnb3, None)

    he = he_k(x4, row, col)
    out = _mlp_call(he, mW0[:H], mW0[H:], mb0, mg0, mlb0,
                    mW1, mb1, mg1, mlb1, mW2, mb2)
    return out.reshape(E)


# speedup vs baseline: 12.9474x; 1.9253x over previous
import functools

import jax
import jax.numpy as jnp
from jax import lax
from jax.experimental import pallas as pl
from jax.experimental.pallas import tpu as pltpu
from jax.experimental.pallas import tpu_sc as plsc

NC, NS, L = 2, 16, 16
NW = NC * NS


def _sc_mesh():
    return plsc.VectorSubcoreMesh(core_axis_name="c", subcore_axis_name="s",
                                  num_cores=NC, num_subcores=NS)


_SC_PARAMS = pltpu.CompilerParams(needs_layout_passes=False)


def _ln(y, g, b):
    m = jnp.mean(y, axis=-1, keepdims=True)
    v = jnp.mean((y - m) ** 2, axis=-1, keepdims=True)
    return (y - m) * lax.rsqrt(v + 1e-5) * g + b


def _make_deg(E, N):
    EPW = E // NW

    @functools.partial(
        pl.kernel,
        out_type=jax.ShapeDtypeStruct((NW * N,), jnp.float32),
        mesh=_sc_mesh(),
        compiler_params=_SC_PARAMS,
        scratch_types=[
            pltpu.VMEM((EPW,), jnp.int32),
            pltpu.VMEM((N,), jnp.float32),
        ],
    )
    def k(col_hbm, out_hbm, colv, hist):
        c = lax.axis_index("c")
        s = lax.axis_index("s")
        wid = s * NC + c
        zero16 = jnp.zeros((L,), jnp.float32)
        ones16 = jnp.ones((L,), jnp.float32)

        def zb(i, carry):
            hist[pl.ds(i * L, L)] = zero16
            return carry

        lax.fori_loop(0, N // L, zb, 0)
        pltpu.sync_copy(col_hbm.at[pl.ds(wid * EPW, EPW)], colv)

        def body(j, carry):
            idx = colv[pl.ds(j * L, L)]
            plsc.addupdate_scatter(hist, [idx], ones16)
            return carry

        lax.fori_loop(0, EPW // L, body, 0)
        pltpu.sync_copy(hist, out_hbm.at[pl.ds(wid * N, N)])

    return k


SC_CH = 125
SC_IB = 2


def _make_scatter(E, N, H):
    EPW = E // NW
    CH = SC_CH
    IB = SC_IB
    NCH = EPW // CH
    NBLK = NCH // IB
    NV = NBLK // 2
    NP = -(-N // (128 * NS)) * (128 * NS)
    NPT = NP // NS

    @functools.partial(
        pl.kernel,
        out_type=jax.ShapeDtypeStruct((NC, NP, H), jnp.float32),
        mesh=_sc_mesh(),
        compiler_params=_SC_PARAMS,
        scratch_types=[
            pltpu.VMEM((IB, CH), jnp.int32),
            pltpu.VMEM((IB, CH), jnp.int32),
            pltpu.VMEM((IB, CH), jnp.int32),
            pltpu.VMEM((IB, CH), jnp.int32),
            pltpu.VMEM((CH, H), jnp.float32),
            pltpu.VMEM((CH, H), jnp.float32),
            pltpu.VMEM_SHARED((NP, H), jnp.float32),
            pltpu.SemaphoreType.DMA,
            pltpu.SemaphoreType.DMA,
            pltpu.SemaphoreType.DMA,
        ],
    )
    def k(g_hbm, row_hbm, col_hbm, z_hbm, out_hbm,
          ri0, ci0, ri1, ci1, bufa, bufb, acc, sema, semb, semi):
        c = lax.axis_index("c")
        s = lax.axis_index("s")
        wid = s * NC + c

        def idx_copy(b, rdst, cdst):
            return (pltpu.make_async_copy(row_hbm.at[wid, b], rdst, semi),
                    pltpu.make_async_copy(col_hbm.at[wid, b], cdst, semi))

        def gather(ridx, j, buf, sem):
            return pltpu.make_async_copy(g_hbm.at[ridx.at[j]], buf, sem)

        pltpu.sync_copy(z_hbm, acc.at[pl.ds(s * NPT, NPT)])
        pltpu.sync_copy(row_hbm.at[wid, 0], ri0)
        pltpu.sync_copy(col_hbm.at[wid, 0], ci0)
        for d in idx_copy(1, ri1, ci1):
            d.start()
        gather(ri0, 0, bufa, sema).start()
        plsc.subcore_barrier()

        def body(v, carry):
            gather(ri0, 1, bufb, semb).start()
            gather(ri0, 0, bufa, sema).wait()
            pltpu.sync_copy(bufa, acc.at[ci0.at[0]], add=True)
            for d in idx_copy(2 * v + 1, ri1, ci1):
                d.wait()
            gather(ri1, 0, bufa, sema).start()
            gather(ri0, 1, bufb, semb).wait()
            pltpu.sync_copy(bufb, acc.at[ci0.at[1]], add=True)

            @pl.when(v < NV - 1)
            def _():
                for d in idx_copy(2 * v + 2, ri0, ci0):
                    d.start()

            gather(ri1, 1, bufb, semb).start()
            gather(ri1, 0, bufa, sema).wait()
            pltpu.sync_copy(bufa, acc.at[ci1.at[0]], add=True)

            @pl.when(v < NV - 1)
            def _():
                for d in idx_copy(2 * v + 2, ri0, ci0):
                    d.wait()
                gather(ri0, 0, bufa, sema).start()

            gather(ri1, 1, bufb, semb).wait()
            pltpu.sync_copy(bufb, acc.at[ci1.at[1]], add=True)

            @pl.when(v < NV - 1)
            def _():
                for d in idx_copy(2 * v + 3, ri1, ci1):
                    d.start()

            return carry

        lax.fori_loop(0, NV, body, 0)
        plsc.subcore_barrier()
        pltpu.sync_copy(acc.at[pl.ds(s * NPT, NPT)],
                        out_hbm.at[c, pl.ds(s * NPT, NPT)])

    return k


HE_CH = 80


def _make_gather_he(E, N, H):
    EPW = E // NW
    CH = HE_CH
    NCH = EPW // CH

    @functools.partial(
        pl.kernel,
        out_type=jax.ShapeDtypeStruct((2, E, H), jnp.float32),
        mesh=_sc_mesh(),
        compiler_params=_SC_PARAMS,
        scratch_types=[
            pltpu.VMEM((NCH, CH), jnp.int32),
            pltpu.VMEM((NCH, CH), jnp.int32),
            pltpu.VMEM((CH, H), jnp.float32),
            pltpu.VMEM((CH, H), jnp.float32),
            pltpu.VMEM((CH, H), jnp.float32),
            pltpu.VMEM((CH, H), jnp.float32),
            pltpu.SemaphoreType.DMA,
            pltpu.SemaphoreType.DMA,
            pltpu.SemaphoreType.DMA,
            pltpu.SemaphoreType.DMA,
        ],
    )
    def k(x4_hbm, src_hbm, dst_hbm, out_hbm, srcv, dstv,
          s0, s1, d0, d1, sems0, sems1, semd0, semd1):
        c = lax.axis_index("c")
        s = lax.axis_index("s")
        wid = s * NC + c
        base = wid * EPW

        pltpu.sync_copy(src_hbm.at[wid], srcv)
        pltpu.sync_copy(dst_hbm.at[wid], dstv)

        def gather(t, buf, sem):
            return pltpu.make_async_copy(x4_hbm.at[srcv.at[t]], buf, sem)

        def gatherd(t, buf, sem):
            return pltpu.make_async_copy(x4_hbm.at[dstv.at[t]], buf, sem)

        gather(0, s0, sems0).start()
        gatherd(0, d0, semd0).start()

        def pair(u, carry):
            t0 = 2 * u
            t1 = t0 + 1
            gather(t1, s1, sems1).start()
            gatherd(t1, d1, semd1).start()
            gather(t0, s0, sems0).wait()
            pltpu.sync_copy(s0, out_hbm.at[0, pl.ds(base + t0 * CH, CH)])
            gatherd(t0, d0, semd0).wait()
            pltpu.sync_copy(d0, out_hbm.at[1, pl.ds(base + t0 * CH, CH)])
            gather(t1 + 1, s0, sems0).start()
            gatherd(t1 + 1, d0, semd0).start()
            gather(t1, s1, sems1).wait()
            pltpu.sync_copy(s1, out_hbm.at[0, pl.ds(base + t1 * CH, CH)])
            gatherd(t1, d1, semd1).wait()
            pltpu.sync_copy(d1, out_hbm.at[1, pl.ds(base + t1 * CH, CH)])
            return carry

        lax.fori_loop(0, NCH // 2, pair, 0)
        tl = NCH - 1
        gather(tl, s0, sems0).wait()
        pltpu.sync_copy(s0, out_hbm.at[0, pl.ds(base + tl * CH, CH)])
        gatherd(tl, d0, semd0).wait()
        pltpu.sync_copy(d0, out_hbm.at[1, pl.ds(base + tl * CH, CH)])

    return k


NB = 1000
EB = 2000


def _prologue_call(degT, x, W0):
    N, D = x.shape
    H = W0.shape[1]

    def body(degT_ref, x_ref, w_ref, dinv_ref, g_ref):
        deg = jnp.sum(degT_ref[...], axis=-1, keepdims=True) + 1.0
        dinv = lax.rsqrt(deg)
        h = jnp.dot(x_ref[...], w_ref[...], preferred_element_type=jnp.float32)
        dinv_ref[...] = dinv
        g_ref[...] = h * dinv

    return pl.pallas_call(
        body,
        grid=(N // NB,),
        in_specs=[
            pl.BlockSpec((NB, NW), lambda i: (i, 0)),
            pl.BlockSpec((NB, D), lambda i: (i, 0)),
            pl.BlockSpec((D, H), lambda i: (0, 0)),
        ],
        out_specs=[
            pl.BlockSpec((NB, 1), lambda i: (i, 0)),
            pl.BlockSpec((NB, H), lambda i: (i, 0)),
        ],
        out_shape=[
            jax.ShapeDtypeStruct((N, 1), jnp.float32),
            jax.ShapeDtypeStruct((N, H), jnp.float32),
        ],
    )(degT, x, W0)


def _layer_call(acc, g, dinv, res, b, lng, lnb, Wn):
    N, H = g.shape
    with_res = res is not None
    with_next = Wn is not None

    def body(*refs):
        it = iter(refs)
        acc_ref = next(it)
        g_ref = next(it)
        dinv_ref = next(it)
        res_ref = next(it) if with_res else None
        b_ref = next(it)
        lng_ref = next(it)
        lnb_ref = next(it)
        w_ref = next(it) if with_next else None
        xn_ref = next(it)
        gn_ref = next(it) if with_next else None

        dinv = dinv_ref[...]
        y = (acc_ref[0] + acc_ref[1] + g_ref[...]) * dinv + b_ref[...]
        y = jnp.maximum(y, 0.0)
        y = _ln(y, lng_ref[...], lnb_ref[...])
        if with_res:
            y = y + res_ref[...]
        xn_ref[...] = y
        if with_next:
            gn_ref[...] = jnp.dot(
                y, w_ref[...], preferred_element_type=jnp.float32) * dinv

    in_specs = [pl.BlockSpec((2, NB, H), lambda i: (0, i, 0)),
                pl.BlockSpec((NB, H), lambda i: (i, 0)),
                pl.BlockSpec((NB, 1), lambda i: (i, 0))]
    ins = [acc, g, dinv]
    if with_res:
        in_specs.append(pl.BlockSpec((NB, H), lambda i: (i, 0)))
        ins.append(res)
    in_specs += [pl.BlockSpec((1, H), lambda i: (0, 0))] * 3
    ins += [b.reshape(1, H), lng.reshape(1, H), lnb.reshape(1, H)]
    if with_next:
        in_specs.append(pl.BlockSpec((H, H), lambda i: (0, 0)))
        ins.append(Wn)

    out_specs = [pl.BlockSpec((NB, H), lambda i: (i, 0))]
    out_shape = [jax.ShapeDtypeStruct((N, H), jnp.float32)]
    if with_next:
        out_specs.append(pl.BlockSpec((NB, H), lambda i: (i, 0)))
        out_shape.append(jax.ShapeDtypeStruct((N, H), jnp.float32))

    out = pl.pallas_call(
        body,
        grid=(N // NB,),
        in_specs=in_specs,
        out_specs=out_specs,
        out_shape=out_shape,
    )(*ins)
    return out if with_next else out[0]


def _mlp_call(he, mW0a, mW0b, mb0, mg0, mlb0, mW1, mb1, mg1, mlb1, mW2, mb2):
    _, E, H = he.shape
    F0 = mW0a.shape[1]
    F1 = mW1.shape[1]

    def body(he_ref, w0a_ref, w0b_ref, b0_ref, g0_ref, lb0_ref,
             w1_ref, b1_ref, g1_ref, lb1_ref, w2_ref, b2_ref, o_ref):
        z = (jnp.dot(he_ref[0], w0a_ref[...], preferred_element_type=jnp.float32)
             + jnp.dot(he_ref[1], w0b_ref[...], preferred_element_type=jnp.float32)
             + b0_ref[...])
        z = jnp.maximum(_ln(z, g0_ref[...], lb0_ref[...]), 0.0)
        t = jnp.dot(z, w1_ref[...], preferred_element_type=jnp.float32) + b1_ref[...]
        t = jnp.maximum(_ln(t, g1_ref[...], lb1_ref[...]), 0.0)
        o_ref[...] = (jnp.dot(t, w2_ref[...], preferred_element_type=jnp.float32)
                      + b2_ref[...])

    return pl.pallas_call(
        body,
        grid=(E // EB,),
        in_specs=[
            pl.BlockSpec((2, EB, H), lambda i: (0, i, 0)),
            pl.BlockSpec((H, F0), lambda i: (0, 0)),
            pl.BlockSpec((H, F0), lambda i: (0, 0)),
            pl.BlockSpec((1, F0), lambda i: (0, 0)),
            pl.BlockSpec((1, F0), lambda i: (0, 0)),
            pl.BlockSpec((1, F0), lambda i: (0, 0)),
            pl.BlockSpec((F0, F1), lambda i: (0, 0)),
            pl.BlockSpec((1, F1), lambda i: (0, 0)),
            pl.BlockSpec((1, F1), lambda i: (0, 0)),
            pl.BlockSpec((1, F1), lambda i: (0, 0)),
            pl.BlockSpec((F1, 1), lambda i: (0, 0)),
            pl.BlockSpec((1, 1), lambda i: (0, 0)),
        ],
        out_specs=[pl.BlockSpec((EB, 1), lambda i: (i, 0))],
        out_shape=[jax.ShapeDtypeStruct((E, 1), jnp.float32)],
    )(he, mW0a, mW0b, mb0.reshape(1, F0), mg0.reshape(1, F0),
      mlb0.reshape(1, F0), mW1, mb1.reshape(1, F1), mg1.reshape(1, F1),
      mlb1.reshape(1, F1), mW2, mb2.reshape(1, 1))[0]


def kernel(x, edge_index, W0, b0, lng0, lnb0, W1, b1, lng1, lnb1,
           W2, b2, lng2, lnb2, W3, b3, lng3, lnb3,
           mW0, mb0, mW1, mb1, mW2, mb2, mg0, mlb0, mg1, mlb1):
    N, D = x.shape
    H = W0.shape[1]
    E = edge_index.shape[1]

    ei = edge_index.astype(jnp.int32)
    row, col = ei[0], ei[1]
    EPW = E // NW
    NBLK = EPW // (SC_CH * SC_IB)
    row_s = row.reshape(NW, NBLK, SC_IB, SC_CH)
    col_s = col.reshape(NW, NBLK, SC_IB, SC_CH)
    row_h = row.reshape(NW, EPW // HE_CH, HE_CH)
    col_h = col.reshape(NW, EPW // HE_CH, HE_CH)
    NP = -(-N // (128 * NS)) * (128 * NS)
    zrows = jnp.zeros((NP // NS, W0.shape[1]), jnp.float32)

    deg_k = _make_deg(E, N)
    scat_k = _make_scatter(E, N, H)
    he_k = _make_gather_he(E, N, H)

    degs = deg_k(col).reshape(NW, N)
    dinv, g0 = _prologue_call(degs.T, x, W0)

    acc0 = scat_k(g0, row_s, col_s, zrows)
    x1, g1 = _layer_call(acc0, g0, dinv, None, b0, lng0, lnb0, W1)
    acc1 = scat_k(g1, row_s, col_s, zrows)
    x2, g2 = _layer_call(acc1, g1, dinv, x, b1, lng1, lnb1, W2)
    acc2 = scat_k(g2, row_s, col_s, zrows)
    x3, g3 = _layer_call(acc2, g2, dinv, None, b2, lng2, lnb2, W3)
    acc3 = scat_k(g3, row_s, col_s, zrows)
    x4 = _layer_call(acc3, g3, dinv, x2, b3, lng3, lnb3, None)

    he = he_k(x4, row_h, col_h)
    out = _mlp_call(he, mW0[:H], mW0[H:], mb0, mg0, mlb0,
                    mW1, mb1, mg1, mlb1, mW2, mb2)
    return out.reshape(E)


# speedup vs baseline: 13.0117x; 1.0050x over previous
import functools

import jax
import jax.numpy as jnp
from jax import lax
from jax.experimental import pallas as pl
from jax.experimental.pallas import tpu as pltpu
from jax.experimental.pallas import tpu_sc as plsc

NC, NS, L = 2, 16, 16
NW = NC * NS


def _sc_mesh():
    return plsc.VectorSubcoreMesh(core_axis_name="c", subcore_axis_name="s",
                                  num_cores=NC, num_subcores=NS)


_SC_PARAMS = pltpu.CompilerParams(needs_layout_passes=False)


def _ln(y, g, b):
    m = jnp.mean(y, axis=-1, keepdims=True)
    v = jnp.mean((y - m) ** 2, axis=-1, keepdims=True)
    return (y - m) * lax.rsqrt(v + 1e-5) * g + b


def _make_deg(E, N):
    EPW = E // NW
    NPd = -(-N // (128 * NS)) * (128 * NS)
    R = NPd // 128

    @functools.partial(
        pl.kernel,
        out_type=jax.ShapeDtypeStruct((NC, R, 128), jnp.float32),
        mesh=_sc_mesh(),
        compiler_params=_SC_PARAMS,
        scratch_types=[
            pltpu.VMEM((EPW,), jnp.int32),
            pltpu.VMEM((R, 128), jnp.float32),
            pltpu.VMEM((R,), jnp.int32),
            pltpu.VMEM_SHARED((R, 128), jnp.float32),
        ],
    )
    def k(col_hbm, out_hbm, colv, hist, rid, acc, ):
        c = lax.axis_index("c")
        s = lax.axis_index("s")
        wid = s * NC + c
        zero16 = jnp.zeros((L,), jnp.float32)
        ones16 = jnp.ones((L,), jnp.float32)
        iota16 = lax.iota(jnp.int32, L)

        def zb(i, carry):
            hist[i // 8, pl.ds((i % 8) * L, L)] = zero16
            return carry

        lax.fori_loop(0, R * 8, zb, 0)
        for j in range(R // L):
            rid[pl.ds(j * L, L)] = iota16 + j * L

        @pl.when(s < R // 8)
        def _():
            pltpu.sync_copy(hist.at[pl.ds(s * 8, 8)], acc.at[pl.ds(s * 8, 8)])

        pltpu.sync_copy(col_hbm.at[pl.ds(wid * EPW, EPW)], colv)
        plsc.subcore_barrier()

        def body(j, carry):
            v = colv[pl.ds(j * L, L)]
            plsc.addupdate_scatter(
                hist, [jnp.right_shift(v, 7), jnp.bitwise_and(v, 127)], ones16)
            return carry

        lax.fori_loop(0, EPW // L, body, 0)
        pltpu.sync_copy(hist, acc.at[rid], add=True)
        plsc.subcore_barrier()

        @pl.when(s < R // 8)
        def _():
            pltpu.sync_copy(acc.at[pl.ds(s * 8, 8)],
                            out_hbm.at[c, pl.ds(s * 8, 8)])

    return k


SC_CH = 125
SC_IB = 2


def _make_scatter(E, N, H):
    EPW = E // NW
    CH = SC_CH
    IB = SC_IB
    NCH = EPW // CH
    NBLK = NCH // IB
    NV = NBLK // 2
    NP = -(-N // (128 * NS)) * (128 * NS)
    NPT = NP // NS

    @functools.partial(
        pl.kernel,
        out_type=jax.ShapeDtypeStruct((NC, NP, H), jnp.float32),
        mesh=_sc_mesh(),
        compiler_params=_SC_PARAMS,
        scratch_types=[
            pltpu.VMEM((IB, CH), jnp.int32),
            pltpu.VMEM((IB, CH), jnp.int32),
            pltpu.VMEM((IB, CH), jnp.int32),
            pltpu.VMEM((IB, CH), jnp.int32),
            pltpu.VMEM((CH, H), jnp.float32),
            pltpu.VMEM((CH, H), jnp.float32),
            pltpu.VMEM_SHARED((NP, H), jnp.float32),
            pltpu.SemaphoreType.DMA,
            pltpu.SemaphoreType.DMA,
            pltpu.SemaphoreType.DMA,
        ],
    )
    def k(g_hbm, row_hbm, col_hbm, z_hbm, out_hbm,
          ri0, ci0, ri1, ci1, bufa, bufb, acc, sema, semb, semi):
        c = lax.axis_index("c")
        s = lax.axis_index("s")
        wid = s * NC + c

        def idx_copy(b, rdst, cdst):
            return (pltpu.make_async_copy(row_hbm.at[wid, b], rdst, semi),
                    pltpu.make_async_copy(col_hbm.at[wid, b], cdst, semi))

        def gather(ridx, j, buf, sem):
            return pltpu.make_async_copy(g_hbm.at[ridx.at[j]], buf, sem)

        pltpu.sync_copy(z_hbm, acc.at[pl.ds(s * NPT, NPT)])
        pltpu.sync_copy(row_hbm.at[wid, 0], ri0)
        pltpu.sync_copy(col_hbm.at[wid, 0], ci0)
        for d in idx_copy(1, ri1, ci1):
            d.start()
        gather(ri0, 0, bufa, sema).start()
        plsc.subcore_barrier()

        def body(v, carry):
            gather(ri0, 1, bufb, semb).start()
            gather(ri0, 0, bufa, sema).wait()
            pltpu.sync_copy(bufa, acc.at[ci0.at[0]], add=True)
            for d in idx_copy(2 * v + 1, ri1, ci1):
                d.wait()
            gather(ri1, 0, bufa, sema).start()
            gather(ri0, 1, bufb, semb).wait()
            pltpu.sync_copy(bufb, acc.at[ci0.at[1]], add=True)

            @pl.when(v < NV - 1)
            def _():
                for d in idx_copy(2 * v + 2, ri0, ci0):
                    d.start()

            gather(ri1, 1, bufb, semb).start()
            gather(ri1, 0, bufa, sema).wait()
            pltpu.sync_copy(bufa, acc.at[ci1.at[0]], add=True)

            @pl.when(v < NV - 1)
            def _():
                for d in idx_copy(2 * v + 2, ri0, ci0):
                    d.wait()
                gather(ri0, 0, bufa, sema).start()

            gather(ri1, 1, bufb, semb).wait()
            pltpu.sync_copy(bufb, acc.at[ci1.at[1]], add=True)

            @pl.when(v < NV - 1)
            def _():
                for d in idx_copy(2 * v + 3, ri1, ci1):
                    d.start()

            return carry

        lax.fori_loop(0, NV, body, 0)
        plsc.subcore_barrier()
        pltpu.sync_copy(acc.at[pl.ds(s * NPT, NPT)],
                        out_hbm.at[c, pl.ds(s * NPT, NPT)])

    return k


HE_CH = 80


def _make_gather_he(E, N, H):
    EPW = E // NW
    CH = HE_CH
    NCH = EPW // CH

    @functools.partial(
        pl.kernel,
        out_type=jax.ShapeDtypeStruct((2, E, H), jnp.float32),
        mesh=_sc_mesh(),
        compiler_params=_SC_PARAMS,
        scratch_types=[
            pltpu.VMEM((NCH, CH), jnp.int32),
            pltpu.VMEM((NCH, CH), jnp.int32),
            pltpu.VMEM((CH, H), jnp.float32),
            pltpu.VMEM((CH, H), jnp.float32),
            pltpu.VMEM((CH, H), jnp.float32),
            pltpu.VMEM((CH, H), jnp.float32),
            pltpu.SemaphoreType.DMA,
            pltpu.SemaphoreType.DMA,
            pltpu.SemaphoreType.DMA,
            pltpu.SemaphoreType.DMA,
        ],
    )
    def k(x4_hbm, src_hbm, dst_hbm, out_hbm, srcv, dstv,
          s0, s1, d0, d1, sems0, sems1, semd0, semd1):
        c = lax.axis_index("c")
        s = lax.axis_index("s")
        wid = s * NC + c
        base = wid * EPW

        pltpu.sync_copy(src_hbm.at[wid], srcv)
        pltpu.sync_copy(dst_hbm.at[wid], dstv)

        def gather(t, buf, sem):
            return pltpu.make_async_copy(x4_hbm.at[srcv.at[t]], buf, sem)

        def gatherd(t, buf, sem):
            return pltpu.make_async_copy(x4_hbm.at[dstv.at[t]], buf, sem)

        gather(0, s0, sems0).start()
        gatherd(0, d0, semd0).start()

        def pair(u, carry):
            t0 = 2 * u
            t1 = t0 + 1
            gather(t1, s1, sems1).start()
            gatherd(t1, d1, semd1).start()
            gather(t0, s0, sems0).wait()
            pltpu.sync_copy(s0, out_hbm.at[0, pl.ds(base + t0 * CH, CH)])
            gatherd(t0, d0, semd0).wait()
            pltpu.sync_copy(d0, out_hbm.at[1, pl.ds(base + t0 * CH, CH)])
            gather(t1 + 1, s0, sems0).start()
            gatherd(t1 + 1, d0, semd0).start()
            gather(t1, s1, sems1).wait()
            pltpu.sync_copy(s1, out_hbm.at[0, pl.ds(base + t1 * CH, CH)])
            gatherd(t1, d1, semd1).wait()
            pltpu.sync_copy(d1, out_hbm.at[1, pl.ds(base + t1 * CH, CH)])
            return carry

        lax.fori_loop(0, NCH // 2, pair, 0)
        tl = NCH - 1
        gather(tl, s0, sems0).wait()
        pltpu.sync_copy(s0, out_hbm.at[0, pl.ds(base + tl * CH, CH)])
        gatherd(tl, d0, semd0).wait()
        pltpu.sync_copy(d0, out_hbm.at[1, pl.ds(base + tl * CH, CH)])

    return k


NB = 1000
EB = 2000


def _prologue_call(degT, x, W0):
    N, D = x.shape
    H = W0.shape[1]
    K = degT.shape[1]

    def body(degT_ref, x_ref, w_ref, dinv_ref, g_ref):
        deg = jnp.sum(degT_ref[...], axis=-1, keepdims=True) + 1.0
        dinv = lax.rsqrt(deg)
        h = jnp.dot(x_ref[...], w_ref[...], preferred_element_type=jnp.float32)
        dinv_ref[...] = dinv
        g_ref[...] = h * dinv

    return pl.pallas_call(
        body,
        grid=(N // NB,),
        in_specs=[
            pl.BlockSpec((NB, K), lambda i: (i, 0)),
            pl.BlockSpec((NB, D), lambda i: (i, 0)),
            pl.BlockSpec((D, H), lambda i: (0, 0)),
        ],
        out_specs=[
            pl.BlockSpec((NB, 1), lambda i: (i, 0)),
            pl.BlockSpec((NB, H), lambda i: (i, 0)),
        ],
        out_shape=[
            jax.ShapeDtypeStruct((N, 1), jnp.float32),
            jax.ShapeDtypeStruct((N, H), jnp.float32),
        ],
    )(degT, x, W0)


def _layer_call(acc, g, dinv, res, b, lng, lnb, Wn):
    N, H = g.shape
    with_res = res is not None
    with_next = Wn is not None

    def body(*refs):
        it = iter(refs)
        acc_ref = next(it)
        g_ref = next(it)
        dinv_ref = next(it)
        res_ref = next(it) if with_res else None
        b_ref = next(it)
        lng_ref = next(it)
        lnb_ref = next(it)
        w_ref = next(it) if with_next else None
        xn_ref = next(it)
        gn_ref = next(it) if with_next else None

        dinv = dinv_ref[...]
        y = (acc_ref[0] + acc_ref[1] + g_ref[...]) * dinv + b_ref[...]
        y = jnp.maximum(y, 0.0)
        y = _ln(y, lng_ref[...], lnb_ref[...])
        if with_res:
            y = y + res_ref[...]
        xn_ref[...] = y
        if with_next:
            gn_ref[...] = jnp.dot(
                y, w_ref[...], preferred_element_type=jnp.float32) * dinv

    in_specs = [pl.BlockSpec((2, NB, H), lambda i: (0, i, 0)),
                pl.BlockSpec((NB, H), lambda i: (i, 0)),
                pl.BlockSpec((NB, 1), lambda i: (i, 0))]
    ins = [acc, g, dinv]
    if with_res:
        in_specs.append(pl.BlockSpec((NB, H), lambda i: (i, 0)))
        ins.append(res)
    in_specs += [pl.BlockSpec((1, H), lambda i: (0, 0))] * 3
    ins += [b.reshape(1, H), lng.reshape(1, H), lnb.reshape(1, H)]
    if with_next:
        in_specs.append(pl.BlockSpec((H, H), lambda i: (0, 0)))
        ins.append(Wn)

    out_specs = [pl.BlockSpec((NB, H), lambda i: (i, 0))]
    out_shape = [jax.ShapeDtypeStruct((N, H), jnp.float32)]
    if with_next:
        out_specs.append(pl.BlockSpec((NB, H), lambda i: (i, 0)))
        out_shape.append(jax.ShapeDtypeStruct((N, H), jnp.float32))

    out = pl.pallas_call(
        body,
        grid=(N // NB,),
        in_specs=in_specs,
        out_specs=out_specs,
        out_shape=out_shape,
    )(*ins)
    return out if with_next else out[0]


def _mlp_call(he, mW0a, mW0b, mb0, mg0, mlb0, mW1, mb1, mg1, mlb1, mW2, mb2):
    _, E, H = he.shape
    F0 = mW0a.shape[1]
    F1 = mW1.shape[1]

    bf = jnp.bfloat16

    def body(he_ref, w0a_ref, w0b_ref, b0_ref, g0_ref, lb0_ref,
             w1_ref, b1_ref, g1_ref, lb1_ref, w2_ref, b2_ref, o_ref):
        z = (jnp.dot(he_ref[0].astype(bf), w0a_ref[...].astype(bf),
                     preferred_element_type=jnp.float32)
             + jnp.dot(he_ref[1].astype(bf), w0b_ref[...].astype(bf),
                       preferred_element_type=jnp.float32)
             + b0_ref[...])
        z = jnp.maximum(_ln(z, g0_ref[...], lb0_ref[...]), 0.0)
        t = jnp.dot(z.astype(bf), w1_ref[...].astype(bf),
                    preferred_element_type=jnp.float32) + b1_ref[...]
        t = jnp.maximum(_ln(t, g1_ref[...], lb1_ref[...]), 0.0)
        o_ref[...] = (jnp.dot(t, w2_ref[...],
                              preferred_element_type=jnp.float32)
                      + b2_ref[...])

    return pl.pallas_call(
        body,
        grid=(E // EB,),
        in_specs=[
            pl.BlockSpec((2, EB, H), lambda i: (0, i, 0)),
            pl.BlockSpec((H, F0), lambda i: (0, 0)),
            pl.BlockSpec((H, F0), lambda i: (0, 0)),
            pl.BlockSpec((1, F0), lambda i: (0, 0)),
            pl.BlockSpec((1, F0), lambda i: (0, 0)),
            pl.BlockSpec((1, F0), lambda i: (0, 0)),
            pl.BlockSpec((F0, F1), lambda i: (0, 0)),
            pl.BlockSpec((1, F1), lambda i: (0, 0)),
            pl.BlockSpec((1, F1), lambda i: (0, 0)),
            pl.BlockSpec((1, F1), lambda i: (0, 0)),
            pl.BlockSpec((F1, 1), lambda i: (0, 0)),
            pl.BlockSpec((1, 1), lambda i: (0, 0)),
        ],
        out_specs=[pl.BlockSpec((EB, 1), lambda i: (i, 0))],
        out_shape=[jax.ShapeDtypeStruct((E, 1), jnp.float32)],
    )(he, mW0a, mW0b, mb0.reshape(1, F0), mg0.reshape(1, F0),
      mlb0.reshape(1, F0), mW1, mb1.reshape(1, F1), mg1.reshape(1, F1),
      mlb1.reshape(1, F1), mW2, mb2.reshape(1, 1))[0]


def kernel(x, edge_index, W0, b0, lng0, lnb0, W1, b1, lng1, lnb1,
           W2, b2, lng2, lnb2, W3, b3, lng3, lnb3,
           mW0, mb0, mW1, mb1, mW2, mb2, mg0, mlb0, mg1, mlb1):
    N, D = x.shape
    H = W0.shape[1]
    E = edge_index.shape[1]

    ei = edge_index.astype(jnp.int32)
    row, col = ei[0], ei[1]
    EPW = E // NW
    NBLK = EPW // (SC_CH * SC_IB)
    row_s = row.reshape(NW, NBLK, SC_IB, SC_CH)
    col_s = col.reshape(NW, NBLK, SC_IB, SC_CH)
    row_h = row.reshape(NW, EPW // HE_CH, HE_CH)
    col_h = col.reshape(NW, EPW // HE_CH, HE_CH)
    NP = -(-N // (128 * NS)) * (128 * NS)
    zrows = jnp.zeros((NP // NS, W0.shape[1]), jnp.float32)

    deg_k = _make_deg(E, N)
    scat_k = _make_scatter(E, N, H)
    he_k = _make_gather_he(E, N, H)

    degs = deg_k(col).reshape(NC, -1)[:, :N]
    dinv, g0 = _prologue_call(degs.T, x, W0)

    acc0 = scat_k(g0, row_s, col_s, zrows)
    x1, g1 = _layer_call(acc0, g0, dinv, None, b0, lng0, lnb0, W1)
    acc1 = scat_k(g1, row_s, col_s, zrows)
    x2, g2 = _layer_call(acc1, g1, dinv, x, b1, lng1, lnb1, W2)
    acc2 = scat_k(g2, row_s, col_s, zrows)
    x3, g3 = _layer_call(acc2, g2, dinv, None, b2, lng2, lnb2, W3)
    acc3 = scat_k(g3, row_s, col_s, zrows)
    x4 = _layer_call(acc3, g3, dinv, x2, b3, lng3, lnb3, None)

    he = he_k(x4, row_h, col_h)
    out = _mlp_call(he, mW0[:H], mW0[H:], mb0, mg0, mlb0,
                    mW1, mb1, mg1, mlb1, mW2, mb2)
    return out.reshape(E)
